# Initial kernel scaffold; baseline (speedup 1.0000x reference)
import jax
import jax.numpy as jnp
from jax.experimental import pallas as pl


def kernel(x, edge_index, W1, b1, W2, b2):
    raise NotImplementedError("write your pallas kernel here")


# speedup vs baseline: 15.6436x; 15.6436x over previous
import functools

import jax
import jax.numpy as jnp
from jax import lax
from jax.experimental import pallas as pl
from jax.experimental.pallas import tpu as pltpu
from jax.experimental.pallas import tpu_sc as plsc

N = 10000
E = 160000
IN_CH = 256
HID = 256
NCLS = 16
NSC = 2
NT = 16
LANES = 16
ROWS = 10240
EPAD = 163840
CK = 128
RB = 1000


def _mesh():
    return plsc.VectorSubcoreMesh(
        core_axis_name="c", subcore_axis_name="s",
        num_cores=NSC, num_subcores=NT)


def _sc_deg(dst_pad, zeros16):
    epw = EPAD // (NSC * NT)
    nchunk = epw // CK
    rpt = ROWS // NT

    @functools.partial(
        pl.kernel,
        out_type=jax.ShapeDtypeStruct((NSC, ROWS, 16), jnp.float32),
        mesh=_mesh(),
        scratch_types=[
            pltpu.VMEM((epw,), jnp.int32),
            pltpu.VMEM((CK,), jnp.int32),
            pltpu.VMEM((CK, 16), jnp.float32),
            pltpu.VMEM_SHARED((ROWS, 16), jnp.float32),
        ],
    )
    def k(dst_hbm, z_hbm, out_hbm, dsl, dbuf, ones, acc):
        c = lax.axis_index("c")
        s = lax.axis_index("s")
        w = c * NT + s

        def fill_ones(i, _):
            ones[i, :] = jnp.ones((LANES,), jnp.float32)
            return 0
        lax.fori_loop(0, CK, fill_ones, 0)

        pltpu.sync_copy(z_hbm.at[pl.ds(s * rpt, rpt)], acc.at[pl.ds(s * rpt, rpt)])
        plsc.subcore_barrier()
        pltpu.sync_copy(dst_hbm.at[pl.ds(w * epw, epw)], dsl)

        def chunk(kk, _):
            def fill(j, _):
                dbuf[pl.ds(j * LANES, LANES)] = dsl[pl.ds(kk * CK + j * LANES, LANES)]
                return 0
            lax.fori_loop(0, CK // LANES, fill, 0)
            pltpu.sync_copy(ones, acc.at[dbuf], add=True)
            return 0
        lax.fori_loop(0, nchunk, chunk, 0)

        plsc.subcore_barrier()
        pltpu.sync_copy(acc.at[pl.ds(s * rpt, rpt)],
                        out_hbm.at[c, pl.ds(s * rpt, rpt)])

    return k(dst_pad, zeros16)


def _sc_agg1(src_pad, dst_pad, y_flat, zeros128):
    ept = EPAD // NT
    nchunk = ept // CK
    rpt = ROWS // NT

    @functools.partial(
        pl.kernel,
        out_type=jax.ShapeDtypeStruct((NSC, ROWS, 128), jnp.float32),
        mesh=_mesh(),
        scratch_types=[
            pltpu.VMEM((ept,), jnp.int32),
            pltpu.VMEM((ept,), jnp.int32),
            pltpu.VMEM((CK,), jnp.int32),
            pltpu.VMEM((CK,), jnp.int32),
            pltpu.VMEM((CK, 128), jnp.float32),
            pltpu.VMEM_SHARED((ROWS, 128), jnp.float32),
            pltpu.SemaphoreType.DMA,
        ],
    )
    def k(src_hbm, dst_hbm, y_hbm, z_hbm, out_hbm,
          ssl, dsl, sbuf, dbuf, rows, acc, sem):
        c = lax.axis_index("c")
        s = lax.axis_index("s")
        off = c * N

        pltpu.sync_copy(z_hbm.at[pl.ds(s * rpt, rpt)], acc.at[pl.ds(s * rpt, rpt)])
        plsc.subcore_barrier()
        pltpu.sync_copy(src_hbm.at[pl.ds(s * ept, ept)], ssl)
        pltpu.sync_copy(dst_hbm.at[pl.ds(s * ept, ept)], dsl)

        def chunk(kk, _):
            base = kk * CK

            def fill(j, _):
                sbuf[pl.ds(j * LANES, LANES)] = (
                    ssl[pl.ds(base + j * LANES, LANES)] + off)
                dbuf[pl.ds(j * LANES, LANES)] = dsl[pl.ds(base + j * LANES, LANES)]
                return 0
            lax.fori_loop(0, CK // LANES, fill, 0)
            pltpu.async_copy(y_hbm.at[sbuf], rows, sem).wait()
            pltpu.sync_copy(rows, acc.at[dbuf], add=True)
            return 0
        lax.fori_loop(0, nchunk, chunk, 0)

        plsc.subcore_barrier()
        pltpu.sync_copy(acc.at[pl.ds(s * rpt, rpt)],
                        out_hbm.at[c, pl.ds(s * rpt, rpt)])

    return k(src_pad, dst_pad, y_flat, zeros128)


def _sc_agg2(src_pad, dst_pad, y2, zeros16):
    epw = EPAD // (NSC * NT)
    nchunk = epw // CK
    rpt = ROWS // NT
    ypt = ROWS // NT

    @functools.partial(
        pl.kernel,
        out_type=jax.ShapeDtypeStruct((NSC, ROWS, 16), jnp.float32),
        mesh=_mesh(),
        scratch_types=[
            pltpu.VMEM((epw,), jnp.int32),
            pltpu.VMEM((epw,), jnp.int32),
            pltpu.VMEM((CK,), jnp.int32),
            pltpu.VMEM((CK,), jnp.int32),
            pltpu.VMEM((CK, 16), jnp.float32),
            pltpu.VMEM_SHARED((ROWS, 16), jnp.float32),
            pltpu.VMEM_SHARED((ROWS, 16), jnp.float32),
            pltpu.SemaphoreType.DMA,
        ],
    )
    def k(src_hbm, dst_hbm, y_hbm, z_hbm, out_hbm,
          ssl, dsl, sbuf, dbuf, rows, acc, ys, sem):
        c = lax.axis_index("c")
        s = lax.axis_index("s")
        w = c * NT + s

        pltpu.sync_copy(z_hbm.at[pl.ds(s * rpt, rpt)], acc.at[pl.ds(s * rpt, rpt)])
        pltpu.sync_copy(y_hbm.at[pl.ds(s * ypt, ypt)], ys.at[pl.ds(s * ypt, ypt)])
        plsc.subcore_barrier()
        pltpu.sync_copy(src_hbm.at[pl.ds(w * epw, epw)], ssl)
        pltpu.sync_copy(dst_hbm.at[pl.ds(w * epw, epw)], dsl)

        def chunk(kk, _):
            base = kk * CK

            def fill(j, _):
                sbuf[pl.ds(j * LANES, LANES)] = ssl[pl.ds(base + j * LANES, LANES)]
                dbuf[pl.ds(j * LANES, LANES)] = dsl[pl.ds(base + j * LANES, LANES)]
                return 0
            lax.fori_loop(0, CK // LANES, fill, 0)
            pltpu.async_copy(ys.at[sbuf], rows, sem).wait()
            pltpu.sync_copy(rows, acc.at[dbuf], add=True)
            return 0
        lax.fori_loop(0, nchunk, chunk, 0)

        plsc.subcore_barrier()
        pltpu.sync_copy(acc.at[pl.ds(s * rpt, rpt)],
                        out_hbm.at[c, pl.ds(s * rpt, rpt)])

    return k(src_pad, dst_pad, y2, zeros16)


def _dinv_from(dp):
    deg = dp[0][:, 0:1] + dp[1][:, 0:1] + 1.0
    return lax.rsqrt(deg)


def _tc_mm1(x, W1, degp):
    def body(x_ref, w_ref, dp_ref, o_ref):
        dinv = _dinv_from(dp_ref[...])
        xw = lax.dot_general(x_ref[...], w_ref[...], (((1,), (0,)), ((), ())),
                             precision=lax.Precision.HIGHEST,
                             preferred_element_type=jnp.float32)
        o_ref[...] = xw * dinv

    return pl.pallas_call(
        body,
        grid=(N // RB, 2),
        in_specs=[
            pl.BlockSpec((RB, IN_CH), lambda i, c: (i, 0)),
            pl.BlockSpec((IN_CH, 128), lambda i, c: (0, c)),
            pl.BlockSpec((2, RB, 16), lambda i, c: (0, i, 0)),
        ],
        out_specs=pl.BlockSpec((RB, 128), lambda i, c: (c * (N // RB) + i, 0)),
        out_shape=jax.ShapeDtypeStruct((2 * N, 128), jnp.float32),
    )(x, W1, degp)


def _tc_comb1(agg1, y1r, degp, b1, W2):
    def body(a_ref, y_ref, dp_ref, b1_ref, w2_ref, o_ref):
        dinv = _dinv_from(dp_ref[...])
        a = a_ref[...]
        y = y_ref[...]
        aggf = jnp.concatenate([a[0], a[1]], axis=1)
        yf = jnp.concatenate([y[0], y[1]], axis=1)
        h = jnp.maximum(dinv * (aggf + yf) + b1_ref[...], 0.0)
        xw2 = lax.dot_general(h, w2_ref[...], (((1,), (0,)), ((), ())),
                              precision=lax.Precision.HIGHEST,
                              preferred_element_type=jnp.float32)
        o_ref[...] = dinv * xw2

    return pl.pallas_call(
        body,
        grid=(N // RB,),
        in_specs=[
            pl.BlockSpec((2, RB, 128), lambda i: (0, i, 0)),
            pl.BlockSpec((2, RB, 128), lambda i: (0, i, 0)),
            pl.BlockSpec((2, RB, 16), lambda i: (0, i, 0)),
            pl.BlockSpec((1, HID), lambda i: (0, 0)),
            pl.BlockSpec((HID, NCLS), lambda i: (0, 0)),
        ],
        out_specs=pl.BlockSpec((RB, NCLS), lambda i: (i, 0)),
        out_shape=jax.ShapeDtypeStruct((ROWS, NCLS), jnp.float32),
    )(agg1, y1r, degp, b1.reshape(1, HID), W2)


def _tc_final(agg2, y2, degp, b2):
    def body(a_ref, y_ref, dp_ref, b2_ref, o_ref):
        dinv = _dinv_from(dp_ref[...])
        a = a_ref[...]
        o = dinv * (a[0] + a[1] + y_ref[...]) + b2_ref[...]
        m = jnp.max(o, axis=1, keepdims=True)
        e = jnp.exp(o - m)
        ssum = jnp.sum(e, axis=1, keepdims=True)
        o_ref[...] = (o - m) - jnp.log(ssum)

    return pl.pallas_call(
        body,
        grid=(N // RB,),
        in_specs=[
            pl.BlockSpec((2, RB, 16), lambda i: (0, i, 0)),
            pl.BlockSpec((RB, NCLS), lambda i: (i, 0)),
            pl.BlockSpec((2, RB, 16), lambda i: (0, i, 0)),
            pl.BlockSpec((1, NCLS), lambda i: (0, 0)),
        ],
        out_specs=pl.BlockSpec((RB, NCLS), lambda i: (i, 0)),
        out_shape=jax.ShapeDtypeStruct((N, NCLS), jnp.float32),
    )(agg2, y2, degp, b2.reshape(1, NCLS))


def kernel(x, edge_index, W1, b1, W2, b2):
    ei = edge_index.astype(jnp.int32)
    src = ei[0]
    dst = ei[1]
    pad = EPAD - E
    padr = jnp.arange(pad, dtype=jnp.int32)
    src_p = jnp.concatenate([src, padr % N])
    dst_p = jnp.concatenate([dst, N + padr % (ROWS - N)])
    z16 = jnp.zeros((ROWS, 16), jnp.float32)
    z128 = jnp.zeros((ROWS, 128), jnp.float32)

    degp = _sc_deg(dst_p, z16)
    y1 = _tc_mm1(x, W1, degp)
    agg1 = _sc_agg1(src_p, dst_p, y1, z128)
    y1r = y1.reshape(NSC, N, 128)
    y2 = _tc_comb1(agg1, y1r, degp, b1, W2)
    agg2 = _sc_agg2(src_p, dst_p, y2, z16)
    return _tc_final(agg2, y2, degp, b2)


# speedup vs baseline: 15.9693x; 1.0208x over previous
import functools

import jax
import jax.numpy as jnp
from jax import lax
from jax.experimental import pallas as pl
from jax.experimental.pallas import tpu as pltpu
from jax.experimental.pallas import tpu_sc as plsc

N = 10000
E = 160000
IN_CH = 256
HID = 256
NCLS = 16
NSC = 2
NT = 16
LANES = 16
ROWS = 10240
EPAD = 163840
CK = 128
NC1 = EPAD // NT // CK
NC2 = EPAD // (NSC * NT) // CK
NBUF = 4
RB = 1000


def _mesh():
    return plsc.VectorSubcoreMesh(
        core_axis_name="c", subcore_axis_name="s",
        num_cores=NSC, num_subcores=NT)


def _sc_deg(dstB, zeros16):
    rpt = ROWS // NT

    @functools.partial(
        pl.kernel,
        out_type=jax.ShapeDtypeStruct((NSC, ROWS, 16), jnp.float32),
        mesh=_mesh(),
        scratch_types=[
            pltpu.VMEM((NC2, CK), jnp.int32),
            pltpu.VMEM((CK, 16), jnp.float32),
            pltpu.VMEM_SHARED((ROWS, 16), jnp.float32),
            pltpu.SemaphoreType.DMA,
        ],
    )
    def k(dst_hbm, z_hbm, out_hbm, dsl, ones, acc, sem):
        c = lax.axis_index("c")
        s = lax.axis_index("s")
        w = c * NT + s

        def fill_ones(i, _):
            ones[i, :] = jnp.ones((LANES,), jnp.float32)
            return 0
        lax.fori_loop(0, CK, fill_ones, 0)

        pltpu.sync_copy(z_hbm.at[pl.ds(s * rpt, rpt)], acc.at[pl.ds(s * rpt, rpt)])
        pltpu.sync_copy(dst_hbm.at[w], dsl)
        plsc.subcore_barrier()

        def fire(kk, _):
            pltpu.sync_copy(ones, acc.at[dsl.at[kk]], add=True)
            return 0
        lax.fori_loop(0, NC2, fire, 0)

        plsc.subcore_barrier()
        pltpu.sync_copy(acc.at[pl.ds(s * rpt, rpt)],
                        out_hbm.at[c, pl.ds(s * rpt, rpt)])

    return k(dstB, zeros16)


def _sc_agg1(srcA, dstA, y_flat, zeros128):
    rpt = ROWS // NT

    @functools.partial(
        pl.kernel,
        out_type=jax.ShapeDtypeStruct((NSC, ROWS, 128), jnp.float32),
        mesh=_mesh(),
        scratch_types=[
            pltpu.VMEM((NC1, CK), jnp.int32),
            pltpu.VMEM((NC1, CK), jnp.int32),
            pltpu.VMEM((CK, 128), jnp.float32),
            pltpu.VMEM((CK, 128), jnp.float32),
            pltpu.VMEM((CK, 128), jnp.float32),
            pltpu.VMEM((CK, 128), jnp.float32),
            pltpu.VMEM_SHARED((ROWS, 128), jnp.float32),
            pltpu.SemaphoreType.DMA,
            pltpu.SemaphoreType.DMA,
            pltpu.SemaphoreType.DMA,
            pltpu.SemaphoreType.DMA,
        ],
    )
    def k(src_hbm, dst_hbm, y_hbm, z_hbm, out_hbm,
          ssl, dsl, r0, r1, r2, r3, acc, m0, m1, m2, m3):
        rows = [r0, r1, r2, r3]
        sems = [m0, m1, m2, m3]
        c = lax.axis_index("c")
        s = lax.axis_index("s")

        pltpu.sync_copy(z_hbm.at[pl.ds(s * rpt, rpt)], acc.at[pl.ds(s * rpt, rpt)])
        pltpu.sync_copy(src_hbm.at[c, s], ssl)
        pltpu.sync_copy(dst_hbm.at[s], dsl)

        plsc.subcore_barrier()

        def outer(kk, _):
            pltpu.async_copy(y_hbm.at[ssl.at[kk]], rows[0], sems[0]).wait()
            pltpu.sync_copy(rows[0], acc.at[dsl.at[kk]], add=True)
            return 0
        lax.fori_loop(0, NC1, outer, 0)

        plsc.subcore_barrier()
        pltpu.sync_copy(acc.at[pl.ds(s * rpt, rpt)],
                        out_hbm.at[c, pl.ds(s * rpt, rpt)])

    return k(srcA, dstA, y_flat, zeros128)


def _sc_agg2(srcB, dstB, y2, zeros16):
    rpt = ROWS // NT

    @functools.partial(
        pl.kernel,
        out_type=jax.ShapeDtypeStruct((NSC, ROWS, 16), jnp.float32),
        mesh=_mesh(),
        scratch_types=[
            pltpu.VMEM((NC2, CK), jnp.int32),
            pltpu.VMEM((NC2, CK), jnp.int32),
            pltpu.VMEM((CK, 16), jnp.float32),
            pltpu.VMEM((CK, 16), jnp.float32),
            pltpu.VMEM((CK, 16), jnp.float32),
            pltpu.VMEM((CK, 16), jnp.float32),
            pltpu.VMEM_SHARED((ROWS, 16), jnp.float32),
            pltpu.VMEM_SHARED((ROWS, 16), jnp.float32),
            pltpu.SemaphoreType.DMA,
            pltpu.SemaphoreType.DMA,
            pltpu.SemaphoreType.DMA,
            pltpu.SemaphoreType.DMA,
        ],
    )
    def k(src_hbm, dst_hbm, y_hbm, z_hbm, out_hbm,
          ssl, dsl, r0, r1, r2, r3, acc, ys, m0, m1, m2, m3):
        rows = [r0, r1, r2, r3]
        sems = [m0, m1, m2, m3]
        c = lax.axis_index("c")
        s = lax.axis_index("s")
        w = c * NT + s

        pltpu.sync_copy(z_hbm.at[pl.ds(s * rpt, rpt)], acc.at[pl.ds(s * rpt, rpt)])
        pltpu.sync_copy(y_hbm.at[pl.ds(s * rpt, rpt)], ys.at[pl.ds(s * rpt, rpt)])
        pltpu.sync_copy(src_hbm.at[w], ssl)
        pltpu.sync_copy(dst_hbm.at[w], dsl)
        plsc.subcore_barrier()

        def outer(kk, _):
            pltpu.async_copy(ys.at[ssl.at[kk]], rows[0], sems[0]).wait()
            pltpu.sync_copy(rows[0], acc.at[dsl.at[kk]], add=True)
            return 0
        lax.fori_loop(0, NC2, outer, 0)

        plsc.subcore_barrier()
        pltpu.sync_copy(acc.at[pl.ds(s * rpt, rpt)],
                        out_hbm.at[c, pl.ds(s * rpt, rpt)])

    return k(srcB, dstB, y2, zeros16)


def _dinv_from(dp):
    deg = dp[0][:, 0:1] + dp[1][:, 0:1] + 1.0
    return lax.rsqrt(deg)


def _tc_mm1(x, W1, degp):
    def body(x_ref, w_ref, dp_ref, o_ref):
        dinv = _dinv_from(dp_ref[...])
        xw = lax.dot_general(x_ref[...], w_ref[...], (((1,), (0,)), ((), ())),
                             precision=lax.Precision.HIGHEST,
                             preferred_element_type=jnp.float32)
        o_ref[...] = xw * dinv

    return pl.pallas_call(
        body,
        grid=(N // RB, 2),
        in_specs=[
            pl.BlockSpec((RB, IN_CH), lambda i, c: (i, 0)),
            pl.BlockSpec((IN_CH, 128), lambda i, c: (0, c)),
            pl.BlockSpec((2, RB, 16), lambda i, c: (0, i, 0)),
        ],
        out_specs=pl.BlockSpec((RB, 128), lambda i, c: (c * (N // RB) + i, 0)),
        out_shape=jax.ShapeDtypeStruct((2 * N, 128), jnp.float32),
    )(x, W1, degp)


def _tc_comb1(agg1, y1r, degp, b1, W2):
    def body(a_ref, y_ref, dp_ref, b1_ref, w2_ref, o_ref):
        dinv = _dinv_from(dp_ref[...])
        a = a_ref[...]
        y = y_ref[...]
        aggf = jnp.concatenate([a[0], a[1]], axis=1)
        yf = jnp.concatenate([y[0], y[1]], axis=1)
        h = jnp.maximum(dinv * (aggf + yf) + b1_ref[...], 0.0)
        xw2 = lax.dot_general(h, w2_ref[...], (((1,), (0,)), ((), ())),
                              precision=lax.Precision.HIGHEST,
                              preferred_element_type=jnp.float32)
        o_ref[...] = dinv * xw2

    return pl.pallas_call(
        body,
        grid=(N // RB,),
        in_specs=[
            pl.BlockSpec((2, RB, 128), lambda i: (0, i, 0)),
            pl.BlockSpec((2, RB, 128), lambda i: (0, i, 0)),
            pl.BlockSpec((2, RB, 16), lambda i: (0, i, 0)),
            pl.BlockSpec((1, HID), lambda i: (0, 0)),
            pl.BlockSpec((HID, NCLS), lambda i: (0, 0)),
        ],
        out_specs=pl.BlockSpec((RB, NCLS), lambda i: (i, 0)),
        out_shape=jax.ShapeDtypeStruct((ROWS, NCLS), jnp.float32),
    )(agg1, y1r, degp, b1.reshape(1, HID), W2)


def _tc_final(agg2, y2, degp, b2):
    def body(a_ref, y_ref, dp_ref, b2_ref, o_ref):
        dinv = _dinv_from(dp_ref[...])
        a = a_ref[...]
        o = dinv * (a[0] + a[1] + y_ref[...]) + b2_ref[...]
        m = jnp.max(o, axis=1, keepdims=True)
        e = jnp.exp(o - m)
        ssum = jnp.sum(e, axis=1, keepdims=True)
        o_ref[...] = (o - m) - jnp.log(ssum)

    return pl.pallas_call(
        body,
        grid=(N // RB,),
        in_specs=[
            pl.BlockSpec((2, RB, 16), lambda i: (0, i, 0)),
            pl.BlockSpec((RB, NCLS), lambda i: (i, 0)),
            pl.BlockSpec((2, RB, 16), lambda i: (0, i, 0)),
            pl.BlockSpec((1, NCLS), lambda i: (0, 0)),
        ],
        out_specs=pl.BlockSpec((RB, NCLS), lambda i: (i, 0)),
        out_shape=jax.ShapeDtypeStruct((N, NCLS), jnp.float32),
    )(agg2, y2, degp, b2.reshape(1, NCLS))


def kernel(x, edge_index, W1, b1, W2, b2):
    ei = edge_index.astype(jnp.int32)
    src = ei[0]
    dst = ei[1]
    pad = EPAD - E
    padr = jnp.arange(pad, dtype=jnp.int32)
    src_p = jnp.concatenate([src, padr % N])
    dst_p = jnp.concatenate([dst, N + padr % (ROWS - N)])
    srcA = jnp.stack([src_p, src_p + N]).reshape(NSC, NT, NC1, CK)
    dstA = dst_p.reshape(NT, NC1, CK)
    srcB = src_p.reshape(NSC * NT, NC2, CK)
    dstB = dst_p.reshape(NSC * NT, NC2, CK)
    z16 = jnp.zeros((ROWS, 16), jnp.float32)
    z128 = jnp.zeros((ROWS, 128), jnp.float32)

    degp = _sc_deg(dstB, z16)
    y1 = _tc_mm1(x, W1, degp)
    agg1 = _sc_agg1(srcA, dstA, y1, z128)
    y1r = y1.reshape(NSC, N, 128)
    y2 = _tc_comb1(agg1, y1r, degp, b1, W2)
    agg2 = _sc_agg2(srcB, dstB, y2, z16)
    return _tc_final(agg2, y2, degp, b2)


# speedup vs baseline: 20.6635x; 1.2940x over previous
import functools

import jax
import jax.numpy as jnp
from jax import lax
from jax.experimental import pallas as pl
from jax.experimental.pallas import tpu as pltpu
from jax.experimental.pallas import tpu_sc as plsc

N = 10000
E = 160000
IN_CH = 256
HID = 256
NCLS = 16
NSC = 2
NT = 16
LANES = 16
ROWS = 10240
EPAD = 163840
CK = 128
NC1 = EPAD // NT // CK
NC2 = EPAD // (NSC * NT) // CK
NBUF = 4
RB = 1000


def _mesh():
    return plsc.VectorSubcoreMesh(
        core_axis_name="c", subcore_axis_name="s",
        num_cores=NSC, num_subcores=NT)


def _sc_deg(pkB, zeros16):
    rpt = ROWS // NT

    @functools.partial(
        pl.kernel,
        out_type=jax.ShapeDtypeStruct((NSC, ROWS, 16), jnp.float32),
        mesh=_mesh(),
        scratch_types=[
            pltpu.VMEM((NC2, CK), jnp.int32),
            pltpu.VMEM((CK,), jnp.int32),
            pltpu.VMEM((CK,), jnp.int32),
            pltpu.VMEM((CK,), jnp.int32),
            pltpu.VMEM((CK,), jnp.int32),
            pltpu.VMEM((CK, 16), jnp.float32),
            pltpu.VMEM_SHARED((ROWS, 16), jnp.float32),
            pltpu.SemaphoreType.DMA,
            pltpu.SemaphoreType.DMA,
            pltpu.SemaphoreType.DMA,
            pltpu.SemaphoreType.DMA,
        ],
    )
    def k(pk_hbm, z_hbm, out_hbm, pks, db0, db1, db2, db3, ones, acc,
          m0, m1, m2, m3):
        c = lax.axis_index("c")
        s = lax.axis_index("s")
        w = c * NT + s
        db = [db0, db1, db2, db3]
        sems = [m0, m1, m2, m3]

        def fill_ones(i, _):
            ones[i, :] = jnp.ones((LANES,), jnp.float32)
            return 0
        lax.fori_loop(0, CK, fill_ones, 0)

        pltpu.sync_copy(z_hbm.at[pl.ds(s * rpt, rpt)], acc.at[pl.ds(s * rpt, rpt)])
        pltpu.sync_copy(pk_hbm.at[w], pks)
        plsc.subcore_barrier()

        def unpack(kk, b):
            def fill(j, _):
                p = pks[kk, pl.ds(j * LANES, LANES)]
                db[b][pl.ds(j * LANES, LANES)] = p >> 14
                return 0
            lax.fori_loop(0, CK // LANES, fill, 0)

        def outer(g, _):
            base = g * 4
            descs = []
            for b in range(4):
                unpack(base + b, b)
                descs.append(
                    pltpu.async_copy(ones, acc.at[db[b]], sems[b], add=True))
            for d in descs:
                d.wait()
            return 0
        lax.fori_loop(0, NC2 // 4, outer, 0)

        plsc.subcore_barrier()
        pltpu.sync_copy(acc.at[pl.ds(s * rpt, rpt)],
                        out_hbm.at[c, pl.ds(s * rpt, rpt)])

    return k(pkB, zeros16)


def _sc_agg1(pkA, y_flat, zeros128):
    rpt = ROWS // NT
    G = 10

    @functools.partial(
        pl.kernel,
        out_type=jax.ShapeDtypeStruct((NSC, ROWS, 128), jnp.float32),
        mesh=_mesh(),
        scratch_types=[
            pltpu.VMEM((NC1, CK), jnp.int32),
            pltpu.VMEM((CK,), jnp.int32),
            pltpu.VMEM((CK,), jnp.int32),
            pltpu.VMEM((CK,), jnp.int32),
            pltpu.VMEM((CK,), jnp.int32),
            pltpu.VMEM((CK, 128), jnp.float32),
            pltpu.VMEM((CK, 128), jnp.float32),
            pltpu.VMEM_SHARED((ROWS, 128), jnp.float32),
            pltpu.SemaphoreType.DMA,
            pltpu.SemaphoreType.DMA,
        ],
    )
    def k(pk_hbm, y_hbm, z_hbm, out_hbm,
          pks, sb0, sb1, db0, db1, r0, r1, acc, m0, m1):
        c = lax.axis_index("c")
        s = lax.axis_index("s")
        off = c * N
        sb = [sb0, sb1]
        db = [db0, db1]
        rows = [r0, r1]
        sems = [m0, m1]

        pltpu.sync_copy(z_hbm.at[pl.ds(s * rpt, rpt)], acc.at[pl.ds(s * rpt, rpt)])
        pltpu.sync_copy(pk_hbm.at[s], pks)
        plsc.subcore_barrier()

        def unpack(kk, b):
            def fill(j, _):
                p = pks[kk, pl.ds(j * LANES, LANES)]
                sb[b][pl.ds(j * LANES, LANES)] = (p & 0x3FFF) + off
                db[b][pl.ds(j * LANES, LANES)] = p >> 14
                return 0
            lax.fori_loop(0, CK // LANES, fill, 0)

        def outer(g, _):
            base = g * G
            descs = [None, None]
            for b in range(2):
                unpack(base + b, b)
                descs[b] = pltpu.async_copy(y_hbm.at[sb[b]], rows[b], sems[b])
            for i in range(G):
                b = i % 2
                descs[b].wait()
                pltpu.sync_copy(rows[b], acc.at[db[b]], add=True)
                if i + 2 < G:
                    unpack(base + i + 2, b)
                    descs[b] = pltpu.async_copy(y_hbm.at[sb[b]], rows[b], sems[b])
            return 0
        lax.fori_loop(0, NC1 // G, outer, 0)

        plsc.subcore_barrier()
        pltpu.sync_copy(acc.at[pl.ds(s * rpt, rpt)],
                        out_hbm.at[c, pl.ds(s * rpt, rpt)])

    return k(pkA, y_flat, zeros128)


def _sc_agg2(pkB, y2, zeros16):
    rpt = ROWS // NT
    G = 10

    @functools.partial(
        pl.kernel,
        out_type=jax.ShapeDtypeStruct((NSC, ROWS, 16), jnp.float32),
        mesh=_mesh(),
        scratch_types=[
            pltpu.VMEM((NC2, CK), jnp.int32),
            pltpu.VMEM((CK,), jnp.int32),
            pltpu.VMEM((CK,), jnp.int32),
            pltpu.VMEM((CK,), jnp.int32),
            pltpu.VMEM((CK,), jnp.int32),
            pltpu.VMEM((CK, 16), jnp.float32),
            pltpu.VMEM((CK, 16), jnp.float32),
            pltpu.VMEM_SHARED((ROWS, 16), jnp.float32),
            pltpu.VMEM_SHARED((ROWS, 16), jnp.float32),
            pltpu.SemaphoreType.DMA,
            pltpu.SemaphoreType.DMA,
        ],
    )
    def k(pk_hbm, y_hbm, z_hbm, out_hbm,
          pks, sb0, sb1, db0, db1, r0, r1, acc, ys, m0, m1):
        c = lax.axis_index("c")
        s = lax.axis_index("s")
        w = c * NT + s
        sb = [sb0, sb1]
        db = [db0, db1]
        rows = [r0, r1]
        sems = [m0, m1]

        pltpu.sync_copy(z_hbm.at[pl.ds(s * rpt, rpt)], acc.at[pl.ds(s * rpt, rpt)])
        pltpu.sync_copy(y_hbm.at[pl.ds(s * rpt, rpt)], ys.at[pl.ds(s * rpt, rpt)])
        pltpu.sync_copy(pk_hbm.at[w], pks)
        plsc.subcore_barrier()

        def unpack(kk, b):
            def fill(j, _):
                p = pks[kk, pl.ds(j * LANES, LANES)]
                sb[b][pl.ds(j * LANES, LANES)] = p & 0x3FFF
                db[b][pl.ds(j * LANES, LANES)] = p >> 14
                return 0
            lax.fori_loop(0, CK // LANES, fill, 0)

        def outer(g, _):
            base = g * G
            descs = [None, None]
            for b in range(2):
                unpack(base + b, b)
                descs[b] = pltpu.async_copy(ys.at[sb[b]], rows[b], sems[b])
            for i in range(G):
                b = i % 2
                descs[b].wait()
                pltpu.sync_copy(rows[b], acc.at[db[b]], add=True)
                if i + 2 < G:
                    unpack(base + i + 2, b)
                    descs[b] = pltpu.async_copy(ys.at[sb[b]], rows[b], sems[b])
            return 0
        lax.fori_loop(0, NC2 // G, outer, 0)

        plsc.subcore_barrier()
        pltpu.sync_copy(acc.at[pl.ds(s * rpt, rpt)],
                        out_hbm.at[c, pl.ds(s * rpt, rpt)])

    return k(pkB, y2, zeros16)


def _dinv_from(dp):
    deg = dp[0][:, 0:1] + dp[1][:, 0:1] + 1.0
    return lax.rsqrt(deg)


def _tc_xw(x, W1):
    def body(x_ref, w_ref, o_ref):
        o_ref[...] = lax.dot_general(x_ref[...], w_ref[...],
                                     (((1,), (0,)), ((), ())),
                                     precision=lax.Precision.HIGHEST,
                                     preferred_element_type=jnp.float32)

    return pl.pallas_call(
        body,
        grid=(N // RB, 2),
        in_specs=[
            pl.BlockSpec((RB, IN_CH), lambda i, c: (i, 0)),
            pl.BlockSpec((IN_CH, 128), lambda i, c: (0, c)),
        ],
        out_specs=pl.BlockSpec((RB, 128), lambda i, c: (c * (N // RB) + i, 0)),
        out_shape=jax.ShapeDtypeStruct((2 * N, 128), jnp.float32),
    )(x, W1)


def _tc_scale(xw, degp):
    def body(x_ref, dp_ref, o_ref):
        dinv = _dinv_from(dp_ref[...])
        x2 = x_ref[...]
        o_ref[...] = x2 * dinv[None]

    return pl.pallas_call(
        body,
        grid=(N // RB,),
        in_specs=[
            pl.BlockSpec((2, RB, 128), lambda i: (0, i, 0)),
            pl.BlockSpec((2, RB, 16), lambda i: (0, i, 0)),
        ],
        out_specs=pl.BlockSpec((2, RB, 128), lambda i: (0, i, 0)),
        out_shape=jax.ShapeDtypeStruct((2, N, 128), jnp.float32),
    )(xw, degp)


def _tc_comb1(agg1, y1r, degp, b1, W2):
    def body(a_ref, y_ref, dp_ref, b1_ref, w2_ref, o_ref):
        dinv = _dinv_from(dp_ref[...])
        a = a_ref[...]
        y = y_ref[...]
        aggf = jnp.concatenate([a[0], a[1]], axis=1)
        yf = jnp.concatenate([y[0], y[1]], axis=1)
        h = jnp.maximum(dinv * (aggf + yf) + b1_ref[...], 0.0)
        xw2 = lax.dot_general(h, w2_ref[...], (((1,), (0,)), ((), ())),
                              precision=lax.Precision.HIGHEST,
                              preferred_element_type=jnp.float32)
        o_ref[...] = dinv * xw2

    return pl.pallas_call(
        body,
        grid=(N // RB,),
        in_specs=[
            pl.BlockSpec((2, RB, 128), lambda i: (0, i, 0)),
            pl.BlockSpec((2, RB, 128), lambda i: (0, i, 0)),
            pl.BlockSpec((2, RB, 16), lambda i: (0, i, 0)),
            pl.BlockSpec((1, HID), lambda i: (0, 0)),
            pl.BlockSpec((HID, NCLS), lambda i: (0, 0)),
        ],
        out_specs=pl.BlockSpec((RB, NCLS), lambda i: (i, 0)),
        out_shape=jax.ShapeDtypeStruct((ROWS, NCLS), jnp.float32),
    )(agg1, y1r, degp, b1.reshape(1, HID), W2)


def _tc_final(agg2, y2, degp, b2):
    def body(a_ref, y_ref, dp_ref, b2_ref, o_ref):
        dinv = _dinv_from(dp_ref[...])
        a = a_ref[...]
        o = dinv * (a[0] + a[1] + y_ref[...]) + b2_ref[...]
        m = jnp.max(o, axis=1, keepdims=True)
        e = jnp.exp(o - m)
        ssum = jnp.sum(e, axis=1, keepdims=True)
        o_ref[...] = (o - m) - jnp.log(ssum)

    return pl.pallas_call(
        body,
        grid=(N // RB,),
        in_specs=[
            pl.BlockSpec((2, RB, 16), lambda i: (0, i, 0)),
            pl.BlockSpec((RB, NCLS), lambda i: (i, 0)),
            pl.BlockSpec((2, RB, 16), lambda i: (0, i, 0)),
            pl.BlockSpec((1, NCLS), lambda i: (0, 0)),
        ],
        out_specs=pl.BlockSpec((RB, NCLS), lambda i: (i, 0)),
        out_shape=jax.ShapeDtypeStruct((N, NCLS), jnp.float32),
    )(agg2, y2, degp, b2.reshape(1, NCLS))


def kernel(x, edge_index, W1, b1, W2, b2):
    ei = edge_index.astype(jnp.int32)
    src = ei[0]
    dst = ei[1]
    pad = EPAD - E
    padr = jnp.arange(pad, dtype=jnp.int32)
    src_p = jnp.concatenate([src, padr % N])
    dst_p = jnp.concatenate([dst, N + padr % (ROWS - N)])
    packed = src_p | (dst_p << 14)
    pkA = packed.reshape(NT, NC1, CK)
    pkB = packed.reshape(NSC * NT, NC2, CK)
    z16 = jnp.zeros((ROWS, 16), jnp.float32)
    z128 = jnp.zeros((ROWS, 128), jnp.float32)

    degp = _sc_deg(pkB, z16)
    xw = _tc_xw(x, W1)
    y1r = _tc_scale(xw.reshape(NSC, N, 128), degp)
    y1 = y1r.reshape(2 * N, 128)
    agg1 = _sc_agg1(pkA, y1, z128)
    y2 = _tc_comb1(agg1, y1r, degp, b1, W2)
    agg2 = _sc_agg2(pkB, y2, z16)
    return _tc_final(agg2, y2, degp, b2)
